# baseline (speedup 1.0000x reference)
import functools

import jax
import jax.numpy as jnp
from jax import lax
from jax.experimental import pallas as pl
from jax.experimental.pallas import tpu as pltpu
from jax.experimental.pallas import tpu_sc as plsc

EMBED_DIM = 64
BATCH = 16384


@functools.cache
def _build():
    info = plsc.get_sparse_core_info()
    NC, NS = info.num_cores, info.num_subcores
    NW = NC * NS
    b_per_w = BATCH // NW
    mesh = plsc.VectorSubcoreMesh(core_axis_name="c", subcore_axis_name="s")

    out_t = jax.ShapeDtypeStruct((BATCH, EMBED_DIM), jnp.float32)

    @functools.partial(
        pl.kernel,
        mesh=mesh,
        out_type=[out_t, out_t, out_t],
        scratch_types=[
            pltpu.VMEM((b_per_w,), jnp.int32),
            pltpu.VMEM((b_per_w, EMBED_DIM), jnp.float32),
            [pltpu.SemaphoreType.DMA] * 8,
        ],
    )
    def lookup(W_u, W_i, W_c, id_u, id_i, id_c, out_u, out_i, out_c,
               idx_v, obuf, sems):
        wid = lax.axis_index("s") * NC + lax.axis_index("c")
        base = wid * b_per_w

        for W2, ids, out in ((W_u, id_u, out_u),
                             (W_i, id_i, out_i),
                             (W_c, id_c, out_c)):
            W3 = W2.reshape(W2.shape[0] // 8, 8, EMBED_DIM)
            pltpu.sync_copy(ids.at[pl.ds(base, b_per_w)], idx_v)

            def fire(g, _, W3=W3):
                v = idx_v[pl.ds(g * 16, 16)]
                for l in range(16):
                    i = v[l]
                    pltpu.async_copy(
                        W3.at[lax.shift_right_logical(i, 3),
                              lax.bitwise_and(i, 7)],
                        obuf.at[g * 16 + l],
                        sems[l % 8],
                    )
                return _

            lax.fori_loop(0, b_per_w // 16, fire, 0)

            for k in range(8):
                pltpu.make_async_copy(
                    W2.at[pl.ds(0, b_per_w // 8)],
                    obuf.at[pl.ds(0, b_per_w // 8)],
                    sems[k],
                ).wait()

            pltpu.sync_copy(obuf, out.at[pl.ds(base, b_per_w)])

    return lookup


def kernel(W_user, W_item, W_category, user_id, item_id, category_id):
    lookup = _build()
    e_user, e_item, e_category = lookup(
        W_user,
        W_item,
        W_category,
        user_id.astype(jnp.int32),
        item_id.astype(jnp.int32),
        category_id.astype(jnp.int32),
    )
    return (e_user, e_item, e_category)
